# baseline (speedup 1.0000x reference)
import functools

import jax
import jax.numpy as jnp
from jax import lax
from jax.experimental import pallas as pl
from jax.experimental.pallas import tpu as pltpu
from jax.experimental.pallas import tpu_sc as plsc

N = 10000
NP = 10240
E = 160000
F_IN = 256
NC, NS, L = 2, 16, 16
CW = 128
RPT = NP // NS

EPAD = 163840
EB = 128
E_PER_CORE = EPAD // NC
E_PER_TILE_DEG = E_PER_CORE // NS
DEG_PAIRS = E_PER_TILE_DEG // (2 * EB)

AB = 128
E_PER_TILE = EPAD // NS
AGG_GROUPS_VAL = E_PER_TILE // (2 * AB)

_MESH = plsc.VectorSubcoreMesh(
    core_axis_name="c", subcore_axis_name="s", num_cores=NC, num_subcores=NS
)


def _deg_body(dst_hbm, ones_hbm, zeros_hbm, deg_hbm, dst_v, ones_v, acc):
    c = lax.axis_index("c")
    t = lax.axis_index("s")
    pltpu.sync_copy(ones_hbm, ones_v)
    pltpu.sync_copy(zeros_hbm.at[pl.ds(t * RPT, RPT)], acc.at[pl.ds(t * RPT, RPT)])
    plsc.subcore_barrier()

    base = c * E_PER_CORE + t * E_PER_TILE_DEG

    def batch(b, carry):
        off = pl.multiple_of(base + b * EB, 8)
        pltpu.sync_copy(dst_hbm.at[pl.ds(off, EB)], dst_v)
        pltpu.sync_copy(ones_v, acc.at[dst_v], add=True)
        return carry

    lax.fori_loop(0, 2 * DEG_PAIRS, batch, 0)
    plsc.subcore_barrier()
    pltpu.sync_copy(
        acc.at[pl.ds(t * RPT, RPT)], deg_hbm.at[c, pl.ds(t * RPT, RPT)]
    )


_deg = pl.kernel(
    _deg_body,
    out_type=jax.ShapeDtypeStruct((NC, NP, CW), jnp.float32),
    mesh=_MESH,
    scratch_types=[
        pltpu.VMEM((EB,), jnp.int32),
        pltpu.VMEM((EB, CW), jnp.float32),
        pltpu.VMEM_SHARED((NP, CW), jnp.float32),
    ],
)


NBUF = 2


def _agg_body(n_chunks, y_hbm, src_hbm, dst_hbm, z_hbm, *scr):
    src_v = scr[0:NBUF]
    srcoff_v = scr[NBUF:2 * NBUF]
    dst_v = scr[2 * NBUF:3 * NBUF]
    rows_v = scr[3 * NBUF:4 * NBUF]
    gsem = scr[4 * NBUF:5 * NBUF]
    ssem = scr[5 * NBUF:6 * NBUF]
    acc = scr[6 * NBUF]
    c = lax.axis_index("c")
    t = lax.axis_index("s")

    def load_idx(eoff, roff, j):
        pltpu.sync_copy(src_hbm.at[pl.ds(eoff, AB)], src_v[j])
        for q in range(AB // L):
            srcoff_v[j][pl.ds(q * L, L)] = src_v[j][pl.ds(q * L, L)] + roff
        pltpu.sync_copy(dst_hbm.at[pl.ds(eoff, AB)], dst_v[j])

    for k in range(n_chunks // NC):
        chunk = c * (n_chunks // NC) + k
        roff = chunk * NP
        pltpu.sync_copy(
            y_hbm.at[pl.ds(roff + t * RPT, RPT)], acc.at[pl.ds(t * RPT, RPT)]
        )
        plsc.subcore_barrier()

        ebase = t * E_PER_TILE
        for j in range(NBUF):
            load_idx(pl.multiple_of(ebase + j * AB, 8), roff, j)
            pltpu.async_copy(y_hbm.at[srcoff_v[j]], rows_v[j], gsem[j])

        def group(g, carry):
            for j in range(NBUF):
                pltpu.make_async_copy(
                    y_hbm.at[srcoff_v[j]], rows_v[j], gsem[j]).wait()
                pltpu.async_copy(
                    rows_v[j], acc.at[dst_v[j]], ssem[j], add=True)
            for j in range(NBUF):
                pltpu.make_async_copy(
                    rows_v[j], acc.at[dst_v[j]], ssem[j]).wait()

                @pl.when(g < AGG_GROUPS_VAL - 1)
                def _():
                    eoff = pl.multiple_of(
                        ebase + ((g + 1) * NBUF + j) * AB, 8)
                    load_idx(eoff, roff, j)
                    pltpu.async_copy(y_hbm.at[srcoff_v[j]], rows_v[j], gsem[j])
            return carry

        lax.fori_loop(0, AGG_GROUPS_VAL, group, 0)
        plsc.subcore_barrier()
        pltpu.sync_copy(
            acc.at[pl.ds(t * RPT, RPT)], z_hbm.at[pl.ds(roff + t * RPT, RPT)]
        )
        plsc.subcore_barrier()


def _make_agg(n_chunks):
    return pl.kernel(
        functools.partial(_agg_body, n_chunks),
        out_type=jax.ShapeDtypeStruct((n_chunks * NP, CW), jnp.float32),
        mesh=_MESH,
        scratch_types=(
            [pltpu.VMEM((AB,), jnp.int32)] * 2
            + [pltpu.VMEM((AB,), jnp.int32)] * 2
            + [pltpu.VMEM((AB,), jnp.int32)] * 2
            + [pltpu.VMEM((AB, CW), jnp.float32)] * 2
            + [pltpu.SemaphoreType.DMA] * 4
            + [pltpu.VMEM_SHARED((NP, CW), jnp.float32)]
        ),
    )


_agg2 = _make_agg(2)
_agg4 = _make_agg(4)


RB = 1024


def _scale_body(deg_ref, x_ref, y_ref, dinv_ref):
    dp = deg_ref[...]
    d = dp[0][:, 0:1] + dp[1][:, 0:1] + 1.0
    dinv = lax.rsqrt(jnp.maximum(d, 1e-12))
    dinv_ref[...] = dinv
    s = x_ref[...] * dinv
    y_ref[0] = s[:, :CW]
    y_ref[1] = s[:, CW:]


_scale = pl.pallas_call(
    _scale_body,
    grid=(NP // RB,),
    in_specs=[
        pl.BlockSpec((NC, RB, CW), lambda i: (0, i, 0)),
        pl.BlockSpec((RB, F_IN), lambda i: (i, 0)),
    ],
    out_specs=[
        pl.BlockSpec((2, RB, CW), lambda i: (0, i, 0)),
        pl.BlockSpec((RB, 1), lambda i: (i, 0)),
    ],
    out_shape=[
        jax.ShapeDtypeStruct((2, NP, CW), jnp.float32),
        jax.ShapeDtypeStruct((NP, 1), jnp.float32),
    ],
)


def _l1_body(z_ref, w_ref, b_ref, dinv_ref, y_ref):
    z = z_ref[...]
    zz = jnp.concatenate([z[0], z[1]], axis=1)
    tt = jnp.dot(zz, w_ref[...], preferred_element_type=jnp.float32)
    dinv = dinv_ref[...]
    h = jnp.maximum(tt * dinv + b_ref[...], 0.0)
    y = h * dinv
    for q in range(4):
        y_ref[q] = y[:, q * CW:(q + 1) * CW]


_l1 = pl.pallas_call(
    _l1_body,
    grid=(NP // RB,),
    in_specs=[
        pl.BlockSpec((2, RB, CW), lambda i: (0, i, 0)),
        pl.BlockSpec((F_IN, 512), lambda i: (0, 0)),
        pl.BlockSpec((1, 512), lambda i: (0, 0)),
        pl.BlockSpec((RB, 1), lambda i: (i, 0)),
    ],
    out_specs=pl.BlockSpec((4, RB, CW), lambda i: (0, i, 0)),
    out_shape=jax.ShapeDtypeStruct((4, NP, CW), jnp.float32),
)


def _mlp_body(z_ref, w2_ref, b2_ref, dinv_ref, wf1_ref, bf1_ref,
              wf2_ref, bf2_ref, wf3_ref, bf3_ref, o_ref):
    z = z_ref[...]
    zz = jnp.concatenate([z[0], z[1], z[2], z[3]], axis=1)
    u = jnp.dot(zz, w2_ref[...], preferred_element_type=jnp.float32)
    h2 = jnp.maximum(u * dinv_ref[...] + b2_ref[...], 0.0)
    h3 = jnp.maximum(
        jnp.dot(h2, wf1_ref[...], preferred_element_type=jnp.float32)
        + bf1_ref[...], 0.0)
    h4 = jnp.maximum(
        jnp.dot(h3, wf2_ref[...], preferred_element_type=jnp.float32)
        + bf2_ref[...], 0.0)
    o_ref[...] = (
        jnp.dot(h4, wf3_ref[...], preferred_element_type=jnp.float32)
        + bf3_ref[...])


_mlp = pl.pallas_call(
    _mlp_body,
    grid=(NP // RB,),
    in_specs=[
        pl.BlockSpec((4, RB, CW), lambda i: (0, i, 0)),
        pl.BlockSpec((512, 512), lambda i: (0, 0)),
        pl.BlockSpec((1, 512), lambda i: (0, 0)),
        pl.BlockSpec((RB, 1), lambda i: (i, 0)),
        pl.BlockSpec((512, 256), lambda i: (0, 0)),
        pl.BlockSpec((1, 256), lambda i: (0, 0)),
        pl.BlockSpec((256, 128), lambda i: (0, 0)),
        pl.BlockSpec((1, 128), lambda i: (0, 0)),
        pl.BlockSpec((128, 256), lambda i: (0, 0)),
        pl.BlockSpec((1, 256), lambda i: (0, 0)),
    ],
    out_specs=pl.BlockSpec((RB, 256), lambda i: (i, 0)),
    out_shape=jax.ShapeDtypeStruct((NP, 256), jnp.float32),
)


@jax.jit
def kernel(x, edge_index, W1, b1, W2, b2, Wf1, bf1, Wf2, bf2, Wf3, bf3):
    pad_rows = N + (jnp.arange(EPAD - E, dtype=jnp.int32) % (NP - N))
    src = jnp.concatenate([edge_index[0].astype(jnp.int32), pad_rows])
    dst = jnp.concatenate([edge_index[1].astype(jnp.int32), pad_rows])

    xp = jnp.pad(x, ((0, NP - N), (0, 0)))

    ones_eb = jnp.ones((EB, CW), jnp.float32)
    zeros_n = jnp.zeros((NP, CW), jnp.float32)
    degp = _deg(dst, ones_eb, zeros_n)
    y1, dinv = _scale(degp, xp)
    z1 = _agg2(y1.reshape(2 * NP, CW), src, dst)
    y2 = _l1(z1.reshape(2, NP, CW), W1, b1.reshape(1, -1), dinv)
    z2 = _agg4(y2.reshape(4 * NP, CW), src, dst)
    out = _mlp(
        z2.reshape(4, NP, CW), W2, b2.reshape(1, -1), dinv,
        Wf1, bf1.reshape(1, -1), Wf2, bf2.reshape(1, -1),
        Wf3, bf3.reshape(1, -1))
    return out[:N]


# speedup vs baseline: 1.0142x; 1.0142x over previous
import functools

import jax
import jax.numpy as jnp
from jax import lax
from jax.experimental import pallas as pl
from jax.experimental.pallas import tpu as pltpu
from jax.experimental.pallas import tpu_sc as plsc

N = 10000
NP = 10240
E = 160000
F_IN = 256
NC, NS, L = 2, 16, 16
CW = 128
RPT = NP // NS

EPAD = 163840
EB = 128
E_PER_CORE = EPAD // NC
E_PER_TILE_DEG = E_PER_CORE // NS
DEG_BATCHES = E_PER_TILE_DEG // EB

AB = 128
E_PER_TILE = EPAD // NS
AGG_PAIRS = E_PER_TILE // (2 * AB)

_MESH = plsc.VectorSubcoreMesh(
    core_axis_name="c", subcore_axis_name="s", num_cores=NC, num_subcores=NS
)


def _deg_body(dst_hbm, ones_hbm, zeros_hbm, deg_hbm, dst_v, ones_v, acc):
    c = lax.axis_index("c")
    t = lax.axis_index("s")
    pltpu.sync_copy(ones_hbm, ones_v)
    pltpu.sync_copy(zeros_hbm.at[pl.ds(t * RPT, RPT)], acc.at[pl.ds(t * RPT, RPT)])
    plsc.subcore_barrier()

    base = c * E_PER_CORE + t * E_PER_TILE_DEG

    def batch(b, carry):
        off = pl.multiple_of(base + b * EB, 8)
        pltpu.sync_copy(dst_hbm.at[pl.ds(off, EB)], dst_v)
        pltpu.sync_copy(ones_v, acc.at[dst_v], add=True)
        return carry

    lax.fori_loop(0, DEG_BATCHES, batch, 0)
    plsc.subcore_barrier()
    pltpu.sync_copy(
        acc.at[pl.ds(t * RPT, RPT)], deg_hbm.at[c, pl.ds(t * RPT, RPT)]
    )


_deg = pl.kernel(
    _deg_body,
    out_type=jax.ShapeDtypeStruct((NC, NP, CW), jnp.float32),
    mesh=_MESH,
    scratch_types=[
        pltpu.VMEM((EB,), jnp.int32),
        pltpu.VMEM((EB, CW), jnp.float32),
        pltpu.VMEM_SHARED((NP, CW), jnp.float32),
    ],
)


def _agg_body(n_chunks, y_hbm, src_hbm, dst_hbm, z_hbm,
              src_a, srcoff_a, dst_a, rows_a, sem_a,
              src_b, srcoff_b, dst_b, rows_b, sem_b, acc):
    c = lax.axis_index("c")
    t = lax.axis_index("s")

    def load_idx(eoff, roff, src_v, srcoff_v, dst_v):
        pltpu.sync_copy(src_hbm.at[pl.ds(eoff, AB)], src_v)
        for j in range(AB // L):
            srcoff_v[pl.ds(j * L, L)] = src_v[pl.ds(j * L, L)] + roff
        pltpu.sync_copy(dst_hbm.at[pl.ds(eoff, AB)], dst_v)

    for k in range(n_chunks // NC):
        chunk = c * (n_chunks // NC) + k
        roff = chunk * NP
        pltpu.sync_copy(
            y_hbm.at[pl.ds(roff + t * RPT, RPT)], acc.at[pl.ds(t * RPT, RPT)]
        )
        plsc.subcore_barrier()

        ebase = t * E_PER_TILE
        load_idx(pl.multiple_of(ebase, 8), roff, src_a, srcoff_a, dst_a)
        pltpu.async_copy(y_hbm.at[srcoff_a], rows_a, sem_a)

        def pair(p, carry):
            eoff_a = pl.multiple_of(ebase + (2 * p) * AB, 8)
            load_idx(eoff_a + AB, roff, src_b, srcoff_b, dst_b)
            pltpu.make_async_copy(y_hbm.at[srcoff_a], rows_a, sem_a).wait()
            pltpu.async_copy(y_hbm.at[srcoff_b], rows_b, sem_b)
            pltpu.sync_copy(rows_a, acc.at[dst_a], add=True)

            @pl.when(p < AGG_PAIRS - 1)
            def _():
                load_idx(eoff_a + 2 * AB, roff, src_a, srcoff_a, dst_a)
                pltpu.async_copy(y_hbm.at[srcoff_a], rows_a, sem_a)

            pltpu.make_async_copy(y_hbm.at[srcoff_b], rows_b, sem_b).wait()
            pltpu.sync_copy(rows_b, acc.at[dst_b], add=True)
            return carry

        lax.fori_loop(0, AGG_PAIRS, pair, 0)
        plsc.subcore_barrier()
        pltpu.sync_copy(
            acc.at[pl.ds(t * RPT, RPT)], z_hbm.at[pl.ds(roff + t * RPT, RPT)]
        )
        plsc.subcore_barrier()


def _make_agg(n_chunks):
    return pl.kernel(
        functools.partial(_agg_body, n_chunks),
        out_type=jax.ShapeDtypeStruct((n_chunks * NP, CW), jnp.float32),
        mesh=_MESH,
        scratch_types=[
            pltpu.VMEM((AB,), jnp.int32),
            pltpu.VMEM((AB,), jnp.int32),
            pltpu.VMEM((AB,), jnp.int32),
            pltpu.VMEM((AB, CW), jnp.float32),
            pltpu.SemaphoreType.DMA,
            pltpu.VMEM((AB,), jnp.int32),
            pltpu.VMEM((AB,), jnp.int32),
            pltpu.VMEM((AB,), jnp.int32),
            pltpu.VMEM((AB, CW), jnp.float32),
            pltpu.SemaphoreType.DMA,
            pltpu.VMEM_SHARED((NP, CW), jnp.float32),
        ],
    )


_agg2 = _make_agg(2)
_agg4 = _make_agg(4)


RB = 1024


def _scale_body(deg_ref, x_ref, y_ref, dinv_ref):
    dp = deg_ref[...]
    d = dp[0][:, 0:1] + dp[1][:, 0:1] + 1.0
    dinv = lax.rsqrt(jnp.maximum(d, 1e-12))
    dinv_ref[...] = dinv
    s = x_ref[...] * dinv
    y_ref[0] = s[:, :CW]
    y_ref[1] = s[:, CW:]


_scale = pl.pallas_call(
    _scale_body,
    grid=(NP // RB,),
    in_specs=[
        pl.BlockSpec((NC, RB, CW), lambda i: (0, i, 0)),
        pl.BlockSpec((RB, F_IN), lambda i: (i, 0)),
    ],
    out_specs=[
        pl.BlockSpec((2, RB, CW), lambda i: (0, i, 0)),
        pl.BlockSpec((RB, 1), lambda i: (i, 0)),
    ],
    out_shape=[
        jax.ShapeDtypeStruct((2, NP, CW), jnp.float32),
        jax.ShapeDtypeStruct((NP, 1), jnp.float32),
    ],
)


def _l1_body(z_ref, w_ref, b_ref, dinv_ref, y_ref):
    z = z_ref[...]
    zz = jnp.concatenate([z[0], z[1]], axis=1)
    tt = jnp.dot(zz, w_ref[...], preferred_element_type=jnp.float32)
    dinv = dinv_ref[...]
    h = jnp.maximum(tt * dinv + b_ref[...], 0.0)
    y = h * dinv
    for q in range(4):
        y_ref[q] = y[:, q * CW:(q + 1) * CW]


_l1 = pl.pallas_call(
    _l1_body,
    grid=(NP // RB,),
    in_specs=[
        pl.BlockSpec((2, RB, CW), lambda i: (0, i, 0)),
        pl.BlockSpec((F_IN, 512), lambda i: (0, 0)),
        pl.BlockSpec((1, 512), lambda i: (0, 0)),
        pl.BlockSpec((RB, 1), lambda i: (i, 0)),
    ],
    out_specs=pl.BlockSpec((4, RB, CW), lambda i: (0, i, 0)),
    out_shape=jax.ShapeDtypeStruct((4, NP, CW), jnp.float32),
)


def _mlp_body(z_ref, w2_ref, b2_ref, dinv_ref, wf1_ref, bf1_ref,
              wf2_ref, bf2_ref, wf3_ref, bf3_ref, o_ref):
    z = z_ref[...]
    zz = jnp.concatenate([z[0], z[1], z[2], z[3]], axis=1)
    u = jnp.dot(zz, w2_ref[...], preferred_element_type=jnp.float32)
    h2 = jnp.maximum(u * dinv_ref[...] + b2_ref[...], 0.0)
    h3 = jnp.maximum(
        jnp.dot(h2, wf1_ref[...], preferred_element_type=jnp.float32)
        + bf1_ref[...], 0.0)
    h4 = jnp.maximum(
        jnp.dot(h3, wf2_ref[...], preferred_element_type=jnp.float32)
        + bf2_ref[...], 0.0)
    o_ref[...] = (
        jnp.dot(h4, wf3_ref[...], preferred_element_type=jnp.float32)
        + bf3_ref[...])


_mlp = pl.pallas_call(
    _mlp_body,
    grid=(NP // RB,),
    in_specs=[
        pl.BlockSpec((4, RB, CW), lambda i: (0, i, 0)),
        pl.BlockSpec((512, 512), lambda i: (0, 0)),
        pl.BlockSpec((1, 512), lambda i: (0, 0)),
        pl.BlockSpec((RB, 1), lambda i: (i, 0)),
        pl.BlockSpec((512, 256), lambda i: (0, 0)),
        pl.BlockSpec((1, 256), lambda i: (0, 0)),
        pl.BlockSpec((256, 128), lambda i: (0, 0)),
        pl.BlockSpec((1, 128), lambda i: (0, 0)),
        pl.BlockSpec((128, 256), lambda i: (0, 0)),
        pl.BlockSpec((1, 256), lambda i: (0, 0)),
    ],
    out_specs=pl.BlockSpec((RB, 256), lambda i: (i, 0)),
    out_shape=jax.ShapeDtypeStruct((NP, 256), jnp.float32),
)


@jax.jit
def kernel(x, edge_index, W1, b1, W2, b2, Wf1, bf1, Wf2, bf2, Wf3, bf3):
    pad_rows = N + (jnp.arange(EPAD - E, dtype=jnp.int32) % (NP - N))
    src = jnp.concatenate([edge_index[0].astype(jnp.int32), pad_rows])
    dst = jnp.concatenate([edge_index[1].astype(jnp.int32), pad_rows])

    xp = jnp.pad(x, ((0, NP - N), (0, 0)))

    ones_eb = jnp.ones((EB, CW), jnp.float32)
    zeros_n = jnp.zeros((NP, CW), jnp.float32)
    degp = _deg(dst, ones_eb, zeros_n)
    y1, dinv = _scale(degp, xp)
    z1 = _agg2(y1.reshape(2 * NP, CW), src, dst)
    y2 = _l1(z1.reshape(2, NP, CW), W1, b1.reshape(1, -1), dinv)
    z2 = _agg4(y2.reshape(4 * NP, CW), src, dst)
    out = _mlp(
        z2.reshape(4, NP, CW), W2, b2.reshape(1, -1), dinv,
        Wf1, bf1.reshape(1, -1), Wf2, bf2.reshape(1, -1),
        Wf3, bf3.reshape(1, -1))
    return out[:N]
